# baseline (speedup 1.0000x reference)
import functools

import jax
import jax.numpy as jnp
from jax import lax
from jax.experimental import pallas as pl
from jax.experimental.pallas import tpu as pltpu
from jax.experimental.pallas import tpu_sc as plsc

R, B, D, K, TOPK = 1024, 32, 128, 16, 4
BD = B * D

RBLK = 128


def _tc_gram_body(hfull_ref, nbr_ref, sel_ref, hhi_ref, hlo_ref):
    i = pl.program_id(0)

    @pl.when(i == 0)
    def _split():
        hf = hfull_ref[...]
        hi = hf.astype(jnp.bfloat16)
        hhi_ref[...] = hi
        hlo_ref[...] = (hf - hi.astype(jnp.float32)).astype(jnp.bfloat16)

    lhs_hi = hhi_ref[pl.ds(i * RBLK, RBLK), :]
    lhs_lo = hlo_ref[pl.ds(i * RBLK, RBLK), :]
    rhs_hi = hhi_ref[...]
    rhs_lo = hlo_ref[...]
    dn = (((1,), (1,)), ((), ()))
    s_blk = (
        lax.dot_general(lhs_hi, rhs_hi, dn,
                        preferred_element_type=jnp.float32)
        + lax.dot_general(lhs_hi, rhs_lo, dn,
                          preferred_element_type=jnp.float32)
        + lax.dot_general(lhs_lo, rhs_hi, dn,
                          preferred_element_type=jnp.float32))
    nbr_blk = nbr_ref[...]
    iota_col = lax.broadcasted_iota(jnp.int32, (RBLK, R), 1)
    sims_cols = []
    for k in range(K):
        colk = nbr_blk[:, k:k + 1]
        simk = jnp.sum(jnp.where(iota_col == colk, s_blk, 0.0),
                       axis=1, keepdims=True)
        sims_cols.append(simk)
    work = jnp.concatenate(sims_cols, axis=1)
    iota_k = lax.broadcasted_iota(jnp.int32, (RBLK, K), 1)
    neg_big = jnp.float32(jnp.finfo(jnp.float32).min)
    sels = []
    for _ in range(TOPK):
        m = jnp.max(work, axis=1, keepdims=True)
        amax = jnp.min(jnp.where(work == m, iota_k, K),
                       axis=1, keepdims=True)
        sels.append(jnp.sum(jnp.where(iota_k == amax, nbr_blk, 0),
                            axis=1, keepdims=True))
        work = jnp.where(iota_k == amax, neg_big, work)
    sel_ref[...] = jnp.concatenate(sels, axis=1)


def _tc_gram(h1, nbr):
    return pl.pallas_call(
        _tc_gram_body,
        grid=(R // RBLK,),
        in_specs=[
            pl.BlockSpec((R, BD), lambda i: (0, 0)),
            pl.BlockSpec((RBLK, K), lambda i: (i, 0)),
        ],
        out_specs=pl.BlockSpec((RBLK, TOPK), lambda i: (i, 0)),
        out_shape=jax.ShapeDtypeStruct((R, TOPK), jnp.int32),
        scratch_shapes=[
            pltpu.VMEM((R, BD), jnp.bfloat16),
            pltpu.VMEM((R, BD), jnp.bfloat16),
        ],
    )(h1, nbr)


NW = 32
RW = R // NW
CH = 2
NCHUNK = RW // CH


def _sc_select_gather():
    mesh = plsc.VectorSubcoreMesh(core_axis_name="c", subcore_axis_name="s")

    @functools.partial(
        pl.kernel,
        mesh=mesh,
        out_type=jax.ShapeDtypeStruct((R, BD), jnp.float32),
        compiler_params=pltpu.CompilerParams(use_tc_tiling_on_sc=False,
                                             needs_layout_passes=False),
        scratch_types=[
            pltpu.VMEM((RW * TOPK,), jnp.int32),
            pltpu.VMEM((CH * TOPK, BD), jnp.float32),
            pltpu.VMEM((CH * TOPK, BD), jnp.float32),
            pltpu.VMEM((CH, BD), jnp.float32),
            pltpu.VMEM((CH, BD), jnp.float32),
            pltpu.SemaphoreType.DMA,
            pltpu.SemaphoreType.DMA,
            pltpu.SemaphoreType.DMA,
            pltpu.SemaphoreType.DMA,
        ],
    )
    def body(sel_hbm, h_hbm, hbar_hbm,
             sel_all, rows_v0, rows_v1, acc_v0, acc_v1,
             g_sem0, g_sem1, o_sem0, o_sem1):
        rows_v = (rows_v0, rows_v1)
        acc_v = (acc_v0, acc_v1)
        g_sem = (g_sem0, g_sem1)
        o_sem = (o_sem0, o_sem1)

        cid = lax.axis_index("c")
        sid = lax.axis_index("s")
        wid = sid * 2 + cid
        base = wid * RW
        pltpu.sync_copy(sel_hbm.at[pl.ds(base * TOPK, RW * TOPK)], sel_all)

        def start_gather(c):
            b = c % 2
            return pltpu.async_copy(
                h_hbm.at[sel_all.at[pl.ds(c * CH * TOPK, CH * TOPK)]],
                rows_v[b], g_sem[b])

        def accum_and_out(c):
            b = c % 2
            rb_v, ac_v = rows_v[b], acc_v[b]

            def accum(j, _):
                off = j * 16
                for rr in range(CH):
                    rb = rr * TOPK
                    acc = (rb_v[rb, pl.ds(off, 16)]
                           + rb_v[rb + 1, pl.ds(off, 16)]
                           + rb_v[rb + 2, pl.ds(off, 16)]
                           + rb_v[rb + 3, pl.ds(off, 16)])
                    ac_v[rr, pl.ds(off, 16)] = acc * 0.25
                return 0
            lax.fori_loop(0, BD // 16, accum, 0)
            return pltpu.async_copy(
                ac_v, hbar_hbm.at[pl.ds(base + c * CH, CH)], o_sem[b])

        g_dma = [None] * NCHUNK
        o_dma = [None] * NCHUNK
        g_dma[0] = start_gather(0)
        for c in range(NCHUNK):
            if c + 1 < NCHUNK:
                g_dma[c + 1] = start_gather(c + 1)
            g_dma[c].wait()
            if c >= 2:
                o_dma[c - 2].wait()
            o_dma[c] = accum_and_out(c)
        o_dma[NCHUNK - 2].wait()
        o_dma[NCHUNK - 1].wait()

    return body


CBLK = 4096


def _tc_merge_body(h2_ref, hbar_ref, wm_ref, wmsg_ref, bmsg_ref, bmrg_ref,
                   out_ref):
    wm1 = wm_ref[:, :D]
    wm2 = wm_ref[:, D:]
    w_eff = lax.dot_general(wm2, wmsg_ref[...], (((1,), (0,)), ((), ())),
                            preferred_element_type=jnp.float32)
    b_eff = bmrg_ref[...] + lax.dot_general(
        bmsg_ref[...], wm2, (((1,), (1,)), ((), ())),
        preferred_element_type=jnp.float32)
    dn = (((1,), (1,)), ((), ()))
    out_ref[...] = (lax.dot_general(h2_ref[...], wm1, dn,
                                    preferred_element_type=jnp.float32)
                    + lax.dot_general(hbar_ref[...], w_eff, dn,
                                      preferred_element_type=jnp.float32)
                    + b_eff)


def _tc_merge(h2, hbar2, w_merge, w_msg, b_msg, b_merge):
    return pl.pallas_call(
        _tc_merge_body,
        grid=(R * B // CBLK,),
        in_specs=[
            pl.BlockSpec((CBLK, D), lambda i: (i, 0)),
            pl.BlockSpec((CBLK, D), lambda i: (i, 0)),
            pl.BlockSpec((D, 2 * D), lambda i: (0, 0)),
            pl.BlockSpec((D, D), lambda i: (0, 0)),
            pl.BlockSpec((1, D), lambda i: (0, 0)),
            pl.BlockSpec((1, D), lambda i: (0, 0)),
        ],
        out_specs=pl.BlockSpec((CBLK, D), lambda i: (i, 0)),
        out_shape=jax.ShapeDtypeStruct((R * B, D), jnp.float32),
    )(h2, hbar2, w_merge, w_msg, b_msg, b_merge)


def kernel(h_by_region, neighbor_indices, W_msg, b_msg, W_merge, b_merge):
    h1 = h_by_region.reshape(R, BD)
    h2 = h_by_region.reshape(R * B, D)
    sel = _tc_gram(h1, neighbor_indices)
    hbar = _sc_select_gather()(sel.reshape(R * TOPK), h1)
    out = _tc_merge(h2, hbar.reshape(R * B, D), W_merge, W_msg,
                    b_msg.reshape(1, D), b_merge.reshape(1, D))
    return out.reshape(R, B, D)


# speedup vs baseline: 1.1795x; 1.1795x over previous
import functools

import jax
import jax.numpy as jnp
from jax import lax
from jax.experimental import pallas as pl
from jax.experimental.pallas import tpu as pltpu
from jax.experimental.pallas import tpu_sc as plsc

R, B, D, K, TOPK = 1024, 32, 128, 16, 4
BD = B * D

RBLK = 128


CBLK2 = 256
NJB = R // CBLK2


def _tc_gram_body(hfull_ref, s_ref, hhi_ref, hlo_ref):
    i = pl.program_id(0)
    jb = pl.program_id(1)

    @pl.when((i == 0) & (jb == 0))
    def _split():
        hf = hfull_ref[...]
        hi = hf.astype(jnp.bfloat16)
        hhi_ref[...] = hi
        hlo_ref[...] = (hf - hi.astype(jnp.float32)).astype(jnp.bfloat16)

    @pl.when(jb >= i // 2)
    def _compute():
        lhs_hi = hhi_ref[pl.ds(i * RBLK, RBLK), :]
        lhs_lo = hlo_ref[pl.ds(i * RBLK, RBLK), :]
        rhs_hi = hhi_ref[pl.ds(jb * CBLK2, CBLK2), :]
        rhs_lo = hlo_ref[pl.ds(jb * CBLK2, CBLK2), :]
        dn = (((1,), (1,)), ((), ()))
        s_ref[...] = (
            lax.dot_general(lhs_hi, rhs_hi, dn,
                            preferred_element_type=jnp.float32)
            + lax.dot_general(lhs_hi, rhs_lo, dn,
                              preferred_element_type=jnp.float32)
            + lax.dot_general(lhs_lo, rhs_hi, dn,
                              preferred_element_type=jnp.float32))


def _tc_gram(h1):
    return pl.pallas_call(
        _tc_gram_body,
        grid=(R // RBLK, NJB),
        in_specs=[
            pl.BlockSpec((R, BD), lambda i, jb: (0, 0)),
        ],
        out_specs=pl.BlockSpec(
            (RBLK, CBLK2), lambda i, jb: (i, jnp.maximum(jb, i // 2))),
        out_shape=jax.ShapeDtypeStruct((R, R), jnp.float32),
        scratch_shapes=[
            pltpu.VMEM((R, BD), jnp.bfloat16),
            pltpu.VMEM((R, BD), jnp.bfloat16),
        ],
    )(h1)


NW = 32
RW = R // NW
CH = 2
NCHUNK = RW // CH


def _sc_select_gather():
    mesh = plsc.VectorSubcoreMesh(core_axis_name="c", subcore_axis_name="s")

    @functools.partial(
        pl.kernel,
        mesh=mesh,
        out_type=jax.ShapeDtypeStruct((R, BD), jnp.float32),
        compiler_params=pltpu.CompilerParams(use_tc_tiling_on_sc=False,
                                             needs_layout_passes=False),
        scratch_types=[
            pltpu.VMEM((RW, K), jnp.int32),
            pltpu.VMEM((CH * K,), jnp.int32),
            pltpu.VMEM((CH * K,), jnp.int32),
            pltpu.VMEM((CH * K,), jnp.int32),
            pltpu.VMEM((CH * K,), jnp.int32),
            pltpu.VMEM((CH * K, 16), jnp.float32),
            pltpu.VMEM((CH * K, 16), jnp.float32),
            pltpu.VMEM((CH * K,), jnp.int32),
            pltpu.VMEM((16,), jnp.int32),
            pltpu.VMEM((16,), jnp.int32),
            pltpu.VMEM((CH * TOPK, BD), jnp.float32),
            pltpu.VMEM((CH * TOPK, BD), jnp.float32),
            pltpu.VMEM((CH, BD), jnp.float32),
            pltpu.VMEM((CH, BD), jnp.float32),
            pltpu.SemaphoreType.DMA,
            pltpu.SemaphoreType.DMA,
            pltpu.SemaphoreType.DMA,
            pltpu.SemaphoreType.DMA,
            pltpu.SemaphoreType.DMA,
            pltpu.SemaphoreType.DMA,
        ],
    )
    def body(s16_hbm, nbr_hbm, h_hbm, hbar_hbm,
             nbr_v, si_v0, si_v1, fl_v0, fl_v1, sbuf0, sbuf1,
             sel_v, idx_v0, idx_v1,
             rows_v0, rows_v1, acc_v0, acc_v1,
             ss_sem0, ss_sem1, g_sem0, g_sem1, o_sem0, o_sem1):
        si_v = (si_v0, si_v1)
        fl_v = (fl_v0, fl_v1)
        sbuf = (sbuf0, sbuf1)
        idx_v = (idx_v0, idx_v1)
        rows_v = (rows_v0, rows_v1)
        acc_v = (acc_v0, acc_v1)
        ss_sem = (ss_sem0, ss_sem1)
        g_sem = (g_sem0, g_sem1)
        o_sem = (o_sem0, o_sem1)

        cid = lax.axis_index("c")
        sid = lax.axis_index("s")
        wid = sid * 2 + cid
        base = wid * RW
        pltpu.sync_copy(nbr_hbm.at[pl.ds(base, RW)], nbr_v)

        lanes = lax.iota(jnp.int32, 16)
        perm = ((lanes & 7) >> 2) * K + (lanes & 3)

        def start_sims(c):
            b = c % 2
            for rr in range(CH):
                cand = nbr_v[c * CH + rr, :]
                rv = jnp.full((16,), base + c * CH + rr, jnp.int32)
                mn = jnp.minimum(rv, cand)
                mx = jnp.maximum(rv, cand)
                f = mn * R + mx
                si_v[b][pl.ds(rr * K, K)] = f >> 4
                fl_v[b][pl.ds(rr * K, K)] = f & 15
            return pltpu.async_copy(
                s16_hbm.at[si_v[b]], sbuf[b], ss_sem[b])

        def start_gather(c):
            b = c % 2
            for rr in range(CH):
                cand = nbr_v[c * CH + rr, :]
                srow = (rr * K) + lanes
                scol = fl_v[b][pl.ds(rr * K, K)]
                sims = plsc.load_gather(sbuf[b], [srow, scol])
                _, order = plsc.sort_key_val(sims, cand, descending=True)
                sel_v[pl.ds(rr * K, K)] = order
            idx_v[b][...] = plsc.load_gather(sel_v, [perm])
            return pltpu.async_copy(
                h_hbm.at[idx_v[b].at[pl.ds(0, CH * TOPK)]],
                rows_v[b], g_sem[b])

        def accum_and_out(c):
            b = c % 2
            rb_v, ac_v = rows_v[b], acc_v[b]

            def accum(j, _):
                off = j * 16
                for rr in range(CH):
                    rb = rr * TOPK
                    acc = (rb_v[rb, pl.ds(off, 16)]
                           + rb_v[rb + 1, pl.ds(off, 16)]
                           + rb_v[rb + 2, pl.ds(off, 16)]
                           + rb_v[rb + 3, pl.ds(off, 16)])
                    ac_v[rr, pl.ds(off, 16)] = acc * 0.25
                return 0
            lax.fori_loop(0, BD // 16, accum, 0)
            return pltpu.async_copy(
                ac_v, hbar_hbm.at[pl.ds(base + c * CH, CH)], o_sem[b])

        s_dma = [None] * NCHUNK
        g_dma = [None] * NCHUNK
        o_dma = [None] * NCHUNK
        s_dma[0] = start_sims(0)
        s_dma[0].wait()
        s_dma[1] = start_sims(1)
        g_dma[0] = start_gather(0)
        for c in range(NCHUNK):
            if c + 1 < NCHUNK:
                s_dma[c + 1].wait()
                if c + 2 < NCHUNK:
                    s_dma[c + 2] = start_sims(c + 2)
                g_dma[c + 1] = start_gather(c + 1)
            g_dma[c].wait()
            if c >= 2:
                o_dma[c - 2].wait()
            o_dma[c] = accum_and_out(c)
        o_dma[NCHUNK - 2].wait()
        o_dma[NCHUNK - 1].wait()

    return body


CBLK = 4096


def _tc_merge_body(h2_ref, hbar_ref, wm_ref, wmsg_ref, bmsg_ref, bmrg_ref,
                   out_ref):
    wm1 = wm_ref[:, :D]
    wm2 = wm_ref[:, D:]
    w_eff = lax.dot_general(wm2, wmsg_ref[...], (((1,), (0,)), ((), ())),
                            preferred_element_type=jnp.float32)
    b_eff = bmrg_ref[...] + lax.dot_general(
        bmsg_ref[...], wm2, (((1,), (1,)), ((), ())),
        preferred_element_type=jnp.float32)
    dn = (((1,), (1,)), ((), ()))
    out_ref[...] = (lax.dot_general(h2_ref[...], wm1, dn,
                                    preferred_element_type=jnp.float32)
                    + lax.dot_general(hbar_ref[...], w_eff, dn,
                                      preferred_element_type=jnp.float32)
                    + b_eff)


def _tc_merge(h2, hbar2, w_merge, w_msg, b_msg, b_merge):
    return pl.pallas_call(
        _tc_merge_body,
        grid=(R * B // CBLK,),
        in_specs=[
            pl.BlockSpec((CBLK, D), lambda i: (i, 0)),
            pl.BlockSpec((CBLK, D), lambda i: (i, 0)),
            pl.BlockSpec((D, 2 * D), lambda i: (0, 0)),
            pl.BlockSpec((D, D), lambda i: (0, 0)),
            pl.BlockSpec((1, D), lambda i: (0, 0)),
            pl.BlockSpec((1, D), lambda i: (0, 0)),
        ],
        out_specs=pl.BlockSpec((CBLK, D), lambda i: (i, 0)),
        out_shape=jax.ShapeDtypeStruct((R * B, D), jnp.float32),
    )(h2, hbar2, w_merge, w_msg, b_msg, b_merge)


def kernel(h_by_region, neighbor_indices, W_msg, b_msg, W_merge, b_merge):
    h1 = h_by_region.reshape(R, BD)
    h2 = h_by_region.reshape(R * B, D)
    s = _tc_gram(h1)
    hbar = _sc_select_gather()(s.reshape(R * R // 16, 16),
                               neighbor_indices, h1)
    out = _tc_merge(h2, hbar.reshape(R * B, D), W_merge, W_msg,
                    b_msg.reshape(1, D), b_merge.reshape(1, D))
    return out.reshape(R, B, D)
